# baseline (speedup 1.0000x reference)
import math

import jax
import jax.numpy as jnp
from jax.experimental import pallas as pl


_F32 = jnp.float32
_BF16 = jnp.float32


def _relu(x):
    return jnp.maximum(x, 0.0)


def _rpad(x):
    s1, s2, _ = x.shape
    x = jnp.concatenate([x[1:2], x, x[s1 - 2:s1 - 1]], axis=0)
    x = jnp.concatenate([x[:, 1:2], x, x[:, s2 - 2:s2 - 1]], axis=1)
    return x


def _pool_mat(s2):
    r = jax.lax.broadcasted_iota(jnp.int32, (s2, s2 // 2), 0)
    c = jax.lax.broadcasted_iota(jnp.int32, (s2, s2 // 2), 1)
    return (r // 2 == c).astype(_F32)


def _up_mat(s2):
    r = jax.lax.broadcasted_iota(jnp.int32, (s2, 2 * s2), 0)
    c = jax.lax.broadcasted_iota(jnp.int32, (s2, 2 * s2), 1)
    return (c // 2 == r).astype(_F32)


def _pool(x):
    s1, s2, ch = x.shape
    x = x.reshape(s1 // 2, 2, s2, ch)
    x = x[:, 0] + x[:, 1]
    xt = jnp.swapaxes(x, 1, 2).reshape((s1 // 2) * ch, s2)
    xt = jnp.dot(xt, _pool_mat(s2), preferred_element_type=_F32)
    xt = xt.reshape(s1 // 2, ch, s2 // 2)
    return jnp.swapaxes(xt, 1, 2) * 0.25


def _up(x):
    s1, s2, ch = x.shape
    x = jnp.broadcast_to(x[:, None], (s1, 2, s2, ch)).reshape(2 * s1, s2, ch)
    xt = jnp.swapaxes(x, 1, 2).reshape(2 * s1 * ch, s2)
    xt = jnp.dot(xt, _up_mat(s2), preferred_element_type=_F32)
    xt = xt.reshape(2 * s1, ch, 2 * s2)
    return jnp.swapaxes(xt, 1, 2)


def _conv_taps(xpad, wt_ref, s1, s2, row_off=0):
    acc = None
    for dx in range(3):
        xsd = xpad[row_off:row_off + s1 + 2, dx:dx + s2, :].astype(_BF16)
        for dy in range(3):
            xs = xsd[dy:dy + s1].reshape(s1 * s2, xsd.shape[-1])
            y = jnp.dot(xs, wt_ref[dy * 3 + dx], preferred_element_type=_F32)
            acc = y if acc is None else acc + y
    return acc


def _enc_body(x_ref, w1_ref, b1_ref, w2_ref, b2_ref, w3_ref, b3_ref,
              rw1_ref, rb1_ref, rw2_ref, rb2_ref,
              vw1_ref, vb1_ref, vw2_ref, vb2_ref, o_ref):
    x = x_ref[...] * 2.0 - 1.0
    h1 = []
    for q in range(4):
        y = _conv_taps(x, w1_ref, 32, 128, row_off=32 * q) + b1_ref[...]
        h1.append(_pool(_relu(y).reshape(32, 128, 128)))
    h = jnp.concatenate(h1, axis=0)
    y = _conv_taps(_rpad(h), w2_ref, 64, 64) + b2_ref[...]
    h = _relu(y).reshape(64, 64, 256)
    y = _conv_taps(_rpad(h), w3_ref, 64, 64) + b3_ref[...]
    h = _pool(_relu(y).reshape(64, 64, 128))

    xp = h.reshape(1024, 128)
    r = jax.lax.broadcasted_iota(jnp.int32, (1024, 8), 0)
    c = jax.lax.broadcasted_iota(jnp.int32, (1024, 8), 1)
    s = jnp.where(c % 2 == 0, r // 32, r % 32)
    ref_pos = s.astype(_F32) * (2.0 / 31.0) - 1.0
    pos = ref_pos
    l8 = jax.lax.broadcasted_iota(jnp.int32, (8, 4), 0)
    b4 = jax.lax.broadcasted_iota(jnp.int32, (8, 4), 1)
    smat = (l8 // 2 == b4).astype(_F32)
    b4e = jax.lax.broadcasted_iota(jnp.int32, (4, 128), 0)
    l128 = jax.lax.broadcasted_iota(jnp.int32, (4, 128), 1)
    emat = (l128 // 32 == b4e).astype(_F32)
    scale = 1.0 / math.sqrt(32.0 ** 2 + 32.0 ** 2)
    for f in range(3):
        xp = _relu(jnp.dot(xp, rw1_ref[...], preferred_element_type=_F32)
                   + rb1_ref[...])
        xp = _relu(jnp.dot(xp, rw2_ref[...], preferred_element_type=_F32)
                   + rb2_ref[...])
        v = _relu(jnp.dot(xp, vw1_ref[...], preferred_element_type=_F32)
                  + vb1_ref[...])
        v = jnp.tanh(jnp.dot(v, vw2_ref[...], preferred_element_type=_F32)
                     + vb2_ref[...])
        pos = pos + v
        d2 = (pos - ref_pos) ** 2
        dist = jnp.dot(d2, smat, preferred_element_type=_F32)
        kd = jnp.exp(-dist * scale)
        kde = jnp.dot(kd, emat, preferred_element_type=_F32)
        o_ref[f] = 1024.0 * kde * xp


def _dec_body(x_ref, w1_ref, b1_ref, w2_ref, b2_ref, w3_ref, b3_ref, o_ref):
    y = _up(x_ref[0])
    y = _conv_taps(_rpad(y), w1_ref, 64, 64) + b1_ref[...]
    h = _relu(y).reshape(64, 64, 256)
    y = _conv_taps(_rpad(h), w2_ref, 64, 64) + b2_ref[...]
    h = _relu(y).reshape(64, 64, 128)
    for q in range(4):
        lo, hi = 16 * q - 1, 16 * q + 17
        xq = h[max(lo, 0):min(hi, 64)]
        if lo < 0:
            xq = jnp.concatenate([h[0:1], xq], axis=0)
        if hi > 64:
            xq = jnp.concatenate([xq, h[63:64]], axis=0)
        y = _up(xq)
        y = jnp.concatenate([y[:, 1:2], y, y[:, 126:127]], axis=1)
        y = _conv_taps(y, w3_ref, 32, 128, row_off=1) + b3_ref[...]
        y = (jnp.tanh(y) + 1.0) * 0.5
        o_ref[0, 32 * q:32 * q + 32] = y.reshape(32, 128, 12)


def _wt(w):
    return jnp.transpose(w, (2, 3, 1, 0)).reshape(9, w.shape[1], w.shape[0])


def _bd(wt, nb):
    eye = jnp.eye(nb, dtype=wt.dtype)
    t, i, o = wt.shape
    out = jnp.einsum('tio,bd->tbido', wt, eye).reshape(t, nb * i, nb * o)
    return out.astype(_BF16)


def _bd2(w, nb):
    eye = jnp.eye(nb, dtype=w.dtype)
    i, o = w.shape
    return jnp.einsum('io,bd->bido', w, eye).reshape(nb * i, nb * o)


def _tile_b(b, nb):
    return jnp.tile(b, nb).reshape(1, nb * b.shape[0])


def kernel(x, enc_w1, enc_b1, enc_w2, enc_b2, enc_w3, enc_b3,
           rule_w1, rule_b1, rule_w2, rule_b2,
           vel_w1, vel_b1, vel_w2, vel_b2,
           dec_w1, dec_b1, dec_w2, dec_b2, dec_w3, dec_b3):
    f32 = _F32

    xp = jnp.transpose(x, (2, 3, 0, 1)).reshape(128, 128, 12)
    xp = jnp.pad(xp, ((1, 1), (1, 1), (0, 0)), mode='reflect')
    enc_args = [
        xp,
        _bd(_wt(enc_w1), 4), _tile_b(enc_b1, 4),
        _bd(_wt(enc_w2), 4), _tile_b(enc_b2, 4),
        _bd(_wt(enc_w3), 4), _tile_b(enc_b3, 4),
        _bd2(rule_w1[:, :, 0].T, 4), _tile_b(rule_b1, 4),
        _bd2(rule_w2[:, :, 0].T, 4), _tile_b(rule_b2, 4),
        _bd2(vel_w1[:, :, 0].T, 4), _tile_b(vel_b1, 4),
        _bd2(vel_w2[:, :, 0].T, 4), _tile_b(vel_b2, 4),
    ]
    frames = pl.pallas_call(
        _enc_body,
        in_specs=[pl.BlockSpec(a.shape, lambda *_, n=a.ndim: (0,) * n)
                  for a in enc_args],
        out_specs=pl.BlockSpec((3, 1024, 128), lambda *_: (0, 0, 0)),
        out_shape=jax.ShapeDtypeStruct((3, 1024, 128), f32),
    )(*enc_args)

    fr = frames.reshape(3, 1024, 4, 32).transpose(2, 0, 1, 3)
    fr = fr.reshape(3, 4, 1024, 32).transpose(0, 2, 1, 3)
    fr = fr.reshape(3, 32, 32, 128)

    dec_w = [_bd(_wt(dec_w1), 4), _tile_b(dec_b1, 4),
             _bd(_wt(dec_w2), 4), _tile_b(dec_b2, 4),
             _bd(_wt(dec_w3), 4), _tile_b(dec_b3, 4)]
    d = pl.pallas_call(
        _dec_body,
        grid=(3,),
        in_specs=[pl.BlockSpec((1, 32, 32, 128), lambda i: (i, 0, 0, 0))]
        + [pl.BlockSpec(w.shape, lambda i, n=w.ndim: (0,) * n)
           for w in dec_w],
        out_specs=pl.BlockSpec((1, 128, 128, 12), lambda i: (i, 0, 0, 0)),
        out_shape=jax.ShapeDtypeStruct((3, 128, 128, 12), f32),
    )(fr, *dec_w)

    d = d.reshape(3, 128, 128, 4, 3).transpose(0, 3, 4, 1, 2)
    dec = d.reshape(4, 3, 3, 128, 128)
    return jnp.concatenate([x[:, None], dec], axis=1)


# speedup vs baseline: 1.2248x; 1.2248x over previous
import math

import jax
import jax.numpy as jnp
from jax.experimental import pallas as pl


_F32 = jnp.float32


def _relu(x):
    return jnp.maximum(x, 0.0)


def _rpad(x):
    s1, s2, _ = x.shape
    x = jnp.concatenate([x[1:2], x, x[s1 - 2:s1 - 1]], axis=0)
    x = jnp.concatenate([x[:, 1:2], x, x[:, s2 - 2:s2 - 1]], axis=1)
    return x


def _pool_mat(s2):
    r = jax.lax.broadcasted_iota(jnp.int32, (s2, s2 // 2), 0)
    c = jax.lax.broadcasted_iota(jnp.int32, (s2, s2 // 2), 1)
    return (r // 2 == c).astype(_F32)


def _up_mat(s2):
    r = jax.lax.broadcasted_iota(jnp.int32, (s2, 2 * s2), 0)
    c = jax.lax.broadcasted_iota(jnp.int32, (s2, 2 * s2), 1)
    return (c // 2 == r).astype(_F32)


def _pool(x):
    s1, s2, ch = x.shape
    x = x.reshape(s1 // 2, 2, s2, ch)
    x = x[:, 0] + x[:, 1]
    xt = jnp.swapaxes(x, 1, 2).reshape((s1 // 2) * ch, s2)
    xt = jnp.dot(xt, _pool_mat(s2), preferred_element_type=_F32)
    xt = xt.reshape(s1 // 2, ch, s2 // 2)
    return jnp.swapaxes(xt, 1, 2) * 0.25


def _up(x):
    s1, s2, ch = x.shape
    x = jnp.broadcast_to(x[:, None], (s1, 2, s2, ch)).reshape(2 * s1, s2, ch)
    xt = jnp.swapaxes(x, 1, 2).reshape(2 * s1 * ch, s2)
    xt = jnp.dot(xt, _up_mat(s2), preferred_element_type=_F32)
    xt = xt.reshape(2 * s1, ch, 2 * s2)
    return jnp.swapaxes(xt, 1, 2)


def _conv_taps(xpad, wt_ref, s1, s2, row_off=0):
    acc = None
    for dx in range(3):
        xsd = xpad[row_off:row_off + s1 + 2, dx:dx + s2, :]
        for dy in range(3):
            xs = xsd[dy:dy + s1].reshape(s1 * s2, xsd.shape[-1])
            y = jnp.dot(xs, wt_ref[dy * 3 + dx], preferred_element_type=_F32)
            acc = y if acc is None else acc + y
    return acc


def _e1_body(x_ref, w_ref, b_ref, o_ref):
    x = x_ref[...] * 2.0 - 1.0
    for q in range(4):
        y = _conv_taps(x, w_ref, 32, 128, row_off=32 * q) + b_ref[...]
        y = _relu(y).reshape(32, 128, 128)
        o_ref[16 * q:16 * q + 16] = _pool(y)


def _e2_body(x_ref, w_ref, b_ref, o_ref):
    y = _conv_taps(_rpad(x_ref[...]), w_ref, 64, 64) + b_ref[...]
    o_ref[...] = _relu(y).reshape(64, 64, 256)


def _e3_body(x_ref, w_ref, b_ref, o_ref):
    y = _conv_taps(_rpad(x_ref[...]), w_ref, 64, 64) + b_ref[...]
    y = _relu(y).reshape(64, 64, 128)
    o_ref[...] = _pool(y)


def _part_body(x_ref, rw1_ref, rb1_ref, rw2_ref, rb2_ref,
               vw1_ref, vb1_ref, vw2_ref, vb2_ref, o_ref):
    xp = x_ref[...]
    r = jax.lax.broadcasted_iota(jnp.int32, (1024, 8), 0)
    c = jax.lax.broadcasted_iota(jnp.int32, (1024, 8), 1)
    s = jnp.where(c % 2 == 0, r // 32, r % 32)
    ref_pos = s.astype(_F32) * (2.0 / 31.0) - 1.0
    pos = ref_pos
    l8 = jax.lax.broadcasted_iota(jnp.int32, (8, 4), 0)
    b4 = jax.lax.broadcasted_iota(jnp.int32, (8, 4), 1)
    smat = (l8 // 2 == b4).astype(_F32)
    b4e = jax.lax.broadcasted_iota(jnp.int32, (4, 128), 0)
    l128 = jax.lax.broadcasted_iota(jnp.int32, (4, 128), 1)
    emat = (l128 // 32 == b4e).astype(_F32)
    li = jax.lax.broadcasted_iota(jnp.int32, (128, 128), 0)
    lo = jax.lax.broadcasted_iota(jnp.int32, (128, 128), 1)
    scale = 1.0 / math.sqrt(32.0 ** 2 + 32.0 ** 2)
    acc = [None, None, None]
    for f in range(3):
        xp = _relu(jnp.dot(xp, rw1_ref[...], preferred_element_type=_F32)
                   + rb1_ref[...])
        xp = _relu(jnp.dot(xp, rw2_ref[...], preferred_element_type=_F32)
                   + rb2_ref[...])
        v = _relu(jnp.dot(xp, vw1_ref[...], preferred_element_type=_F32)
                  + vb1_ref[...])
        v = jnp.tanh(jnp.dot(v, vw2_ref[...], preferred_element_type=_F32)
                     + vb2_ref[...])
        pos = pos + v
        d2 = (pos - ref_pos) ** 2
        dist = jnp.dot(d2, smat, preferred_element_type=_F32)
        kd = jnp.exp(-dist * scale)
        kde = jnp.dot(kd, emat, preferred_element_type=_F32)
        frame = 1024.0 * kde * xp
        for g in range(3):
            perm = ((li % 32 == lo % 32)
                    & (3 * (li // 32) + f == 4 * g + lo // 32)).astype(_F32)
            y = jnp.dot(frame, perm, preferred_element_type=_F32)
            acc[g] = y if acc[g] is None else acc[g] + y
    for g in range(3):
        o_ref[g] = acc[g]


def _d1_body(x_ref, w_ref, b_ref, o_ref):
    y = _up(x_ref[0])
    y = _conv_taps(_rpad(y), w_ref, 64, 64) + b_ref[...]
    o_ref[0] = _relu(y).reshape(64, 64, 256)


def _d2_body(x_ref, w_ref, b_ref, o_ref):
    y = _conv_taps(_rpad(x_ref[0]), w_ref, 64, 64) + b_ref[...]
    o_ref[0] = _relu(y).reshape(64, 64, 128)


def _d3_body(x_ref, w_ref, b_ref, o_ref):
    x = x_ref[0]
    for q in range(4):
        lo, hi = 16 * q - 1, 16 * q + 17
        xq = x[max(lo, 0):min(hi, 64)]
        if lo < 0:
            xq = jnp.concatenate([x[0:1], xq], axis=0)
        if hi > 64:
            xq = jnp.concatenate([xq, x[63:64]], axis=0)
        y = _up(xq)
        y = jnp.concatenate([y[:, 1:2], y, y[:, 126:127]], axis=1)
        y = _conv_taps(y, w_ref, 32, 128, row_off=1) + b_ref[...]
        y = (jnp.tanh(y) + 1.0) * 0.5
        o_ref[0, 32 * q:32 * q + 32] = y.reshape(32, 128, 12)


def _wt(w):
    return jnp.transpose(w, (2, 3, 1, 0)).reshape(9, w.shape[1], w.shape[0])


def _bd(wt, nb):
    eye = jnp.eye(nb, dtype=wt.dtype)
    t, i, o = wt.shape
    return jnp.einsum('tio,bd->tbido', wt, eye).reshape(t, nb * i, nb * o)


def _bd2(w, nb):
    eye = jnp.eye(nb, dtype=w.dtype)
    i, o = w.shape
    return jnp.einsum('io,bd->bido', w, eye).reshape(nb * i, nb * o)


def _tile_b(b, nb):
    return jnp.tile(b, nb).reshape(1, nb * b.shape[0])


def _full_call(body, args, out_sd):
    return pl.pallas_call(
        body,
        in_specs=[pl.BlockSpec(a.shape, lambda *_, n=a.ndim: (0,) * n)
                  for a in args],
        out_specs=pl.BlockSpec(out_sd.shape,
                               lambda *_, n=len(out_sd.shape): (0,) * n),
        out_shape=out_sd,
    )(*args)


def _grid_call(body, x, wt, b, out_sd):
    n = x.shape[0]
    return pl.pallas_call(
        body,
        grid=(n,),
        in_specs=[
            pl.BlockSpec((1,) + x.shape[1:], lambda i: (i, 0, 0, 0)),
            pl.BlockSpec(wt.shape, lambda i: (0, 0, 0)),
            pl.BlockSpec(b.shape, lambda i: (0, 0)),
        ],
        out_specs=pl.BlockSpec((1,) + out_sd.shape[1:],
                               lambda i: (i, 0, 0, 0)),
        out_shape=out_sd,
    )(x, wt, b)


def kernel(x, enc_w1, enc_b1, enc_w2, enc_b2, enc_w3, enc_b3,
           rule_w1, rule_b1, rule_w2, rule_b2,
           vel_w1, vel_b1, vel_w2, vel_b2,
           dec_w1, dec_b1, dec_w2, dec_b2, dec_w3, dec_b3):
    f32 = _F32

    xp = jnp.transpose(x, (2, 3, 0, 1)).reshape(128, 128, 12)
    xp = jnp.pad(xp, ((1, 1), (1, 1), (0, 0)), mode='reflect')
    h = _full_call(_e1_body, [xp, _bd(_wt(enc_w1), 4), _tile_b(enc_b1, 4)],
                   jax.ShapeDtypeStruct((64, 64, 128), f32))
    h = _full_call(_e2_body, [h, _bd(_wt(enc_w2), 4), _tile_b(enc_b2, 4)],
                   jax.ShapeDtypeStruct((64, 64, 256), f32))
    h = _full_call(_e3_body, [h, _bd(_wt(enc_w3), 4), _tile_b(enc_b3, 4)],
                   jax.ShapeDtypeStruct((32, 32, 128), f32))

    pw = [_bd2(rule_w1[:, :, 0].T, 4), _tile_b(rule_b1, 4),
          _bd2(rule_w2[:, :, 0].T, 4), _tile_b(rule_b2, 4),
          _bd2(vel_w1[:, :, 0].T, 4), _tile_b(vel_b1, 4),
          _bd2(vel_w2[:, :, 0].T, 4), _tile_b(vel_b2, 4)]
    fr = _full_call(_part_body, [h.reshape(1024, 128)] + pw,
                    jax.ShapeDtypeStruct((3, 1024, 128), f32))
    fr = fr.reshape(3, 32, 32, 128)

    d = _grid_call(_d1_body, fr, _bd(_wt(dec_w1), 4), _tile_b(dec_b1, 4),
                   jax.ShapeDtypeStruct((3, 64, 64, 256), f32))
    d = _grid_call(_d2_body, d, _bd(_wt(dec_w2), 4), _tile_b(dec_b2, 4),
                   jax.ShapeDtypeStruct((3, 64, 64, 128), f32))
    d = _grid_call(_d3_body, d, _bd(_wt(dec_w3), 4), _tile_b(dec_b3, 4),
                   jax.ShapeDtypeStruct((3, 128, 128, 12), f32))

    d = d.reshape(3, 128, 128, 4, 3).transpose(0, 3, 4, 1, 2)
    dec = d.reshape(4, 3, 3, 128, 128)
    return jnp.concatenate([x[:, None], dec], axis=1)


# speedup vs baseline: 1.3536x; 1.1051x over previous
import math

import jax
import jax.numpy as jnp
from jax.experimental import pallas as pl


_F32 = jnp.float32


def _relu(x):
    return jnp.maximum(x, 0.0)


def _rpad(x):
    s1, s2, _ = x.shape
    x = jnp.concatenate([x[1:2], x, x[s1 - 2:s1 - 1]], axis=0)
    x = jnp.concatenate([x[:, 1:2], x, x[:, s2 - 2:s2 - 1]], axis=1)
    return x


def _pool_mat(s2):
    r = jax.lax.broadcasted_iota(jnp.int32, (s2, s2 // 2), 0)
    c = jax.lax.broadcasted_iota(jnp.int32, (s2, s2 // 2), 1)
    return (r // 2 == c).astype(_F32)


def _up_mat(s2):
    r = jax.lax.broadcasted_iota(jnp.int32, (s2, 2 * s2), 0)
    c = jax.lax.broadcasted_iota(jnp.int32, (s2, 2 * s2), 1)
    return (c // 2 == r).astype(_F32)


def _pool(x):
    s1, s2, ch = x.shape
    x = x.reshape(s1 // 2, 2, s2, ch)
    x = x[:, 0] + x[:, 1]
    xt = jnp.swapaxes(x, 1, 2).reshape((s1 // 2) * ch, s2)
    xt = jnp.dot(xt, _pool_mat(s2), preferred_element_type=_F32)
    xt = xt.reshape(s1 // 2, ch, s2 // 2)
    return jnp.swapaxes(xt, 1, 2) * 0.25


def _up(x):
    s1, s2, ch = x.shape
    x = jnp.broadcast_to(x[:, None], (s1, 2, s2, ch)).reshape(2 * s1, s2, ch)
    xt = jnp.swapaxes(x, 1, 2).reshape(2 * s1 * ch, s2)
    xt = jnp.dot(xt, _up_mat(s2), preferred_element_type=_F32)
    xt = xt.reshape(2 * s1, ch, 2 * s2)
    return jnp.swapaxes(xt, 1, 2)


def _conv_taps(xpad, wt_ref, s1, s2, row_off=0):
    acc = None
    for dx in range(3):
        xsd = xpad[row_off:row_off + s1 + 2, dx:dx + s2, :]
        for dy in range(3):
            xs = xsd[dy:dy + s1].reshape(s1 * s2, xsd.shape[-1])
            y = jnp.dot(xs, wt_ref[dy * 3 + dx], preferred_element_type=_F32)
            acc = y if acc is None else acc + y
    return acc


def _e1_body(x_ref, w_ref, b_ref, o_ref):
    x = x_ref[...]
    for q in range(4):
        y = _conv_taps(x, w_ref, 32, 128, row_off=32 * q) + b_ref[...]
        y = _relu(y).reshape(32, 128, 128)
        o_ref[16 * q:16 * q + 16] = _pool(y)


def _e2_body(x_ref, w_ref, b_ref, o_ref):
    y = _conv_taps(_rpad(x_ref[...]), w_ref, 64, 64) + b_ref[...]
    o_ref[...] = _relu(y).reshape(64, 64, 256)


def _e3_body(x_ref, w_ref, b_ref, o_ref):
    y = _conv_taps(_rpad(x_ref[...]), w_ref, 64, 64) + b_ref[...]
    y = _relu(y).reshape(64, 64, 128)
    o_ref[...] = _pool(y)


def _part_body(x_ref, rw1_ref, rb1_ref, rw2_ref, rb2_ref,
               vw1_ref, vb1_ref, vw2_ref, vb2_ref, o_ref):
    xp = x_ref[...]
    r = jax.lax.broadcasted_iota(jnp.int32, (1024, 8), 0)
    c = jax.lax.broadcasted_iota(jnp.int32, (1024, 8), 1)
    s = jnp.where(c % 2 == 0, r // 32, r % 32)
    ref_pos = s.astype(_F32) * (2.0 / 31.0) - 1.0
    pos = ref_pos
    l8 = jax.lax.broadcasted_iota(jnp.int32, (8, 4), 0)
    b4 = jax.lax.broadcasted_iota(jnp.int32, (8, 4), 1)
    smat = (l8 // 2 == b4).astype(_F32)
    b4e = jax.lax.broadcasted_iota(jnp.int32, (4, 128), 0)
    l128 = jax.lax.broadcasted_iota(jnp.int32, (4, 128), 1)
    emat = (l128 // 32 == b4e).astype(_F32)
    li = jax.lax.broadcasted_iota(jnp.int32, (128, 128), 0)
    lo = jax.lax.broadcasted_iota(jnp.int32, (128, 128), 1)
    scale = 1.0 / math.sqrt(32.0 ** 2 + 32.0 ** 2)
    acc = [None, None, None]
    for f in range(3):
        xp = _relu(jnp.dot(xp, rw1_ref[...], preferred_element_type=_F32)
                   + rb1_ref[...])
        xp = _relu(jnp.dot(xp, rw2_ref[...], preferred_element_type=_F32)
                   + rb2_ref[...])
        v = _relu(jnp.dot(xp, vw1_ref[...], preferred_element_type=_F32)
                  + vb1_ref[...])
        v = jnp.tanh(jnp.dot(v, vw2_ref[...], preferred_element_type=_F32)
                     + vb2_ref[...])
        pos = pos + v
        d2 = (pos - ref_pos) ** 2
        dist = jnp.dot(d2, smat, preferred_element_type=_F32)
        kd = jnp.exp(-dist * scale)
        kde = jnp.dot(kd, emat, preferred_element_type=_F32)
        frame = 1024.0 * kde * xp
        for g in range(3):
            perm = ((li % 32 == lo % 32)
                    & (3 * (li // 32) + f == 4 * g + lo // 32)).astype(_F32)
            y = jnp.dot(frame, perm, preferred_element_type=_F32)
            acc[g] = y if acc[g] is None else acc[g] + y
    for g in range(3):
        o_ref[g] = acc[g]


def _parity_conv(x, w_ref, b_ref, n_out, act):
    s = x.shape[0]
    xe = jnp.concatenate([x[0:1], x, x[s - 1:s]], axis=0)
    xe = jnp.concatenate([xe[:, 0:1], xe, xe[:, s - 1:s]], axis=1)
    cols = [xe[:, j:j + s, :] for j in range(3)]
    pieces = []
    for pb in range(2):
        rowpar = []
        for pa in range(2):
            acc = None
            for ry in range(2):
                for rx in range(2):
                    t = ((pa * 2 + pb) * 2 + ry) * 2 + rx
                    xs = cols[pb + rx][pa + ry:pa + ry + s]
                    xs = xs.reshape(s * s, xs.shape[-1])
                    y = jnp.dot(xs, w_ref[t], preferred_element_type=_F32)
                    acc = y if acc is None else acc + y
            y = act(acc + b_ref[...])
            rowpar.append(y.reshape(s, 1, s, n_out))
        pieces.append(jnp.concatenate(rowpar, axis=1).reshape(2 * s, s, n_out))
    return jnp.concatenate(pieces, axis=-1)


def _d1_body(x_ref, w_ref, b_ref, o_ref):
    o_ref[0] = _parity_conv(x_ref[0], w_ref, b_ref, 256, _relu)


def _d2_body(x_ref, w_ref, b_ref, o_ref):
    y = _conv_taps(_rpad(x_ref[0]), w_ref, 64, 64) + b_ref[...]
    o_ref[0] = _relu(y).reshape(64, 64, 128)


def _out_act(y):
    return (jnp.tanh(y) + 1.0) * 0.5


def _d3_body(x_ref, w_ref, b_ref, o_ref):
    o_ref[0] = _parity_conv(x_ref[0], w_ref, b_ref, 12, _out_act)


def _wt(w):
    return jnp.transpose(w, (2, 3, 1, 0)).reshape(9, w.shape[1], w.shape[0])


def _parity_w(w):
    a = jnp.array([[[1., 0., 0.], [0., 1., 1.]],
                   [[1., 1., 0.], [0., 0., 1.]]], dtype=w.dtype)
    w4 = jnp.transpose(w, (2, 3, 1, 0))
    w2 = jnp.einsum('ard,bse,deio->abrsio', a, a, w4)
    return w2.reshape(16, w.shape[1], w.shape[0])


def _bd(wt, nb):
    eye = jnp.eye(nb, dtype=wt.dtype)
    t, i, o = wt.shape
    return jnp.einsum('tio,bd->tbido', wt, eye).reshape(t, nb * i, nb * o)


def _bd2(w, nb):
    eye = jnp.eye(nb, dtype=w.dtype)
    i, o = w.shape
    return jnp.einsum('io,bd->bido', w, eye).reshape(nb * i, nb * o)


def _tile_b(b, nb):
    return jnp.tile(b, nb).reshape(1, nb * b.shape[0])


def _full_call(body, args, out_sd):
    return pl.pallas_call(
        body,
        in_specs=[pl.BlockSpec(a.shape, lambda *_, n=a.ndim: (0,) * n)
                  for a in args],
        out_specs=pl.BlockSpec(out_sd.shape,
                               lambda *_, n=len(out_sd.shape): (0,) * n),
        out_shape=out_sd,
    )(*args)


def _grid_call(body, x, wt, b, out_sd):
    n = x.shape[0]
    return pl.pallas_call(
        body,
        grid=(n,),
        in_specs=[
            pl.BlockSpec((1,) + x.shape[1:], lambda i: (i, 0, 0, 0)),
            pl.BlockSpec(wt.shape, lambda i: (0, 0, 0)),
            pl.BlockSpec(b.shape, lambda i: (0, 0)),
        ],
        out_specs=pl.BlockSpec((1,) + out_sd.shape[1:],
                               lambda i: (i, 0, 0, 0)),
        out_shape=out_sd,
    )(x, wt, b)


def kernel(x, enc_w1, enc_b1, enc_w2, enc_b2, enc_w3, enc_b3,
           rule_w1, rule_b1, rule_w2, rule_b2,
           vel_w1, vel_b1, vel_w2, vel_b2,
           dec_w1, dec_b1, dec_w2, dec_b2, dec_w3, dec_b3):
    f32 = _F32

    xp = jnp.transpose(x, (2, 3, 0, 1)).reshape(128, 128, 12)
    xp = jnp.pad(xp, ((1, 1), (1, 1), (0, 0)), mode='reflect')
    b1_eff = enc_b1 - jnp.sum(enc_w1, axis=(1, 2, 3))
    h = _full_call(_e1_body,
                   [xp, _bd(_wt(enc_w1) * 2.0, 4), _tile_b(b1_eff, 4)],
                   jax.ShapeDtypeStruct((64, 64, 128), f32))
    h = _full_call(_e2_body, [h, _bd(_wt(enc_w2), 4), _tile_b(enc_b2, 4)],
                   jax.ShapeDtypeStruct((64, 64, 256), f32))
    h = _full_call(_e3_body, [h, _bd(_wt(enc_w3), 4), _tile_b(enc_b3, 4)],
                   jax.ShapeDtypeStruct((32, 32, 128), f32))

    pw = [_bd2(rule_w1[:, :, 0].T, 4), _tile_b(rule_b1, 4),
          _bd2(rule_w2[:, :, 0].T, 4), _tile_b(rule_b2, 4),
          _bd2(vel_w1[:, :, 0].T, 4), _tile_b(vel_b1, 4),
          _bd2(vel_w2[:, :, 0].T, 4), _tile_b(vel_b2, 4)]
    fr = _full_call(_part_body, [h.reshape(1024, 128)] + pw,
                    jax.ShapeDtypeStruct((3, 1024, 128), f32))
    fr = fr.reshape(3, 32, 32, 128)

    d = _grid_call(_d1_body, fr, _bd(_parity_w(dec_w1), 4),
                   _tile_b(dec_b1, 4),
                   jax.ShapeDtypeStruct((3, 64, 32, 512), f32))
    d = d.reshape(3, 64, 64, 256)
    d = _grid_call(_d2_body, d, _bd(_wt(dec_w2), 4), _tile_b(dec_b2, 4),
                   jax.ShapeDtypeStruct((3, 64, 64, 128), f32))
    d = _grid_call(_d3_body, d, _bd(_parity_w(dec_w3), 4),
                   _tile_b(dec_b3, 4),
                   jax.ShapeDtypeStruct((3, 128, 64, 24), f32))
    d = d.reshape(3, 128, 128, 12)

    d = d.reshape(3, 128, 128, 4, 3).transpose(0, 3, 4, 1, 2)
    dec = d.reshape(4, 3, 3, 128, 128)
    return jnp.concatenate([x[:, None], dec], axis=1)
